# baseline (speedup 1.0000x reference)
import functools

import jax
import jax.numpy as jnp
from jax import lax
from jax.experimental import pallas as pl
from jax.experimental.pallas import tpu as pltpu
from jax.experimental.pallas import tpu_sc as plsc

_N = 10000
_E = 320000
_DIN = 128
_DIM = 256
_G = 128

_NC = 2
_NS = 16
_NW = _NC * _NS
_EW = _E // _NW
_C = 128
_NCHUNK = -(-_EW // _C)
_EWP = _NCHUNK * _C
_PAD = _EWP - _EW
_NA = _N + 8
_RPT = 624
_TAIL = _N - _NS * _RPT

@functools.cache
def _edge_agg_fn():
    mesh = plsc.VectorSubcoreMesh(core_axis_name="c", subcore_axis_name="s")

    @functools.partial(
        pl.kernel,
        mesh=mesh,
        out_type=jax.ShapeDtypeStruct((_NC * _N, _DIN), jnp.float32),
        scratch_types=[
            pltpu.VMEM((_C,), jnp.int32),
            pltpu.VMEM((_C,), jnp.int32),
            pltpu.VMEM((_NCHUNK, _C), jnp.int32),
            pltpu.VMEM((_C, _DIN), jnp.float32),
            pltpu.VMEM((_C, _DIN), jnp.float32),
            pltpu.VMEM_SHARED((_NA, _DIN), jnp.float32),
            pltpu.SemaphoreType.DMA,
            pltpu.SemaphoreType.DMA,
            pltpu.SemaphoreType.DMA,
            pltpu.SemaphoreType.DMA,
        ],
    )
    def _edge_agg(src_hbm, dst_hbm, x_hbm, out_hbm,
                  sc0, sc1, dst_v, rows0_v, rows1_v, agg_sh,
                  si0, si1, sg0, sg1):
        srcc = (sc0, sc1)
        rows = (rows0_v, rows1_v)
        semi = (si0, si1)
        semg = (sg0, sg1)
        cid = lax.axis_index("c")
        sid = lax.axis_index("s")
        wid = cid * _NS + sid

        zvec = jnp.zeros((16,), jnp.float32)

        def _zfill(k, carry):
            r = k // (_DIN // 16)
            col = (k % (_DIN // 16)) * 16
            rows0_v[r, pl.ds(col, 16)] = zvec
            return carry

        lax.fori_loop(0, _C * (_DIN // 16), _zfill, 0)

        def _zcopy(k, carry):
            pltpu.sync_copy(rows0_v,
                            agg_sh.at[pl.ds(sid * _RPT + k * _C, _C)])
            return carry

        lax.fori_loop(0, _RPT // _C, _zcopy, 0)
        pltpu.sync_copy(rows0_v.at[pl.ds(0, _RPT - (_RPT // _C) * _C)],
                        agg_sh.at[pl.ds(sid * _RPT + (_RPT // _C) * _C,
                                        _RPT - (_RPT // _C) * _C)])

        @pl.when(sid == _NS - 1)
        def _ztail():
            pltpu.sync_copy(rows0_v.at[pl.ds(0, _TAIL)],
                            agg_sh.at[pl.ds(_NS * _RPT, _TAIL)])

        plsc.subcore_barrier()

        pltpu.sync_copy(dst_hbm.at[wid], dst_v)
        ebase = wid * _EWP

        def _idx_load(j, k):
            pltpu.async_copy(src_hbm.at[pl.ds(ebase + j * _C, _C)],
                             srcc[k], semi[k])

        def _gather(j, k):
            pltpu.make_async_copy(src_hbm.at[pl.ds(ebase + j * _C, _C)],
                                  srcc[k], semi[k]).wait()
            pltpu.async_copy(x_hbm.at[srcc[k]], rows[k], semg[k])

        _idx_load(0, 0)
        _idx_load(1, 1)
        _gather(0, 0)

        def _body(j, carry):
            for k in range(2):
                @pl.when(j % 2 == k)
                def _slot(k=k):
                    pltpu.make_async_copy(x_hbm.at[srcc[k]],
                                          rows[k], semg[k]).wait()

                    @pl.when(j + 1 < _NCHUNK)
                    def _pfg():
                        _gather(j + 1, 1 - k)

                    @pl.when(j + 2 < _NCHUNK)
                    def _pfi():
                        _idx_load(j + 2, k)

                    pltpu.sync_copy(rows[k], agg_sh.at[dst_v.at[j]],
                                    add=True)

            return carry

        lax.fori_loop(0, _NCHUNK, _body, 0)
        plsc.subcore_barrier()

        pltpu.sync_copy(agg_sh.at[pl.ds(sid * _RPT, _RPT)],
                        out_hbm.at[pl.ds(cid * _N + sid * _RPT, _RPT)])

        @pl.when(sid == _NS - 1)
        def _ptail():
            pltpu.sync_copy(agg_sh.at[pl.ds(_NS * _RPT, _TAIL)],
                            out_hbm.at[pl.ds(cid * _N + _NS * _RPT, _TAIL)])

    return _edge_agg


_BLK = 400
_NSTEP = _N // _BLK


def _tc_body(x_ref, a0_ref, a1_ref, batch_ref, w1_ref, b1_ref, w2_ref, b2_ref,
             sc_ref, sh_ref, f1w_ref, f1b_ref, f2w_ref, f2b_ref,
             out_ref, acc_ref):
    i = pl.program_id(0)
    h = x_ref[...] + a0_ref[...] + a1_ref[...]
    h1 = jnp.maximum(
        jnp.dot(h, w1_ref[...], preferred_element_type=jnp.float32)
        + b1_ref[...], 0.0)
    h2 = (jnp.dot(h1, w2_ref[...], preferred_element_type=jnp.float32)
          + b2_ref[...])
    h2 = jnp.maximum(h2, 0.0) * sc_ref[...] + sh_ref[...]

    b_row = batch_ref[0, :, :]
    ohT = (jnp.broadcast_to(b_row, (_G, _BLK))
           == lax.broadcasted_iota(jnp.int32, (_G, _BLK), 0)
           ).astype(jnp.float32)
    pooled = jnp.dot(ohT, h2, preferred_element_type=jnp.float32)

    @pl.when(i == 0)
    def _init():
        acc_ref[...] = jnp.zeros_like(acc_ref)

    acc_ref[...] += pooled

    @pl.when(i == _NSTEP - 1)
    def _head():
        g = acc_ref[...]
        gg = jnp.maximum(
            jnp.dot(g, f1w_ref[...], preferred_element_type=jnp.float32)
            + f1b_ref[...], 0.0)
        out_ref[...] = (jnp.dot(gg, f2w_ref[...],
                                preferred_element_type=jnp.float32)
                        + f2b_ref[...])


_tc_call = pl.pallas_call(
    _tc_body,
    grid=(_NSTEP,),
    in_specs=[
        pl.BlockSpec((_BLK, _DIN), lambda i: (i, 0)),
        pl.BlockSpec((_BLK, _DIN), lambda i: (i, 0)),
        pl.BlockSpec((_BLK, _DIN),
                     lambda i: (i + _N // _BLK, 0)),
        pl.BlockSpec((1, 1, _BLK), lambda i: (i, 0, 0)),
        pl.BlockSpec((_DIN, _DIM), lambda i: (0, 0)),
        pl.BlockSpec((1, _DIM), lambda i: (0, 0)),
        pl.BlockSpec((_DIM, _DIM), lambda i: (0, 0)),
        pl.BlockSpec((1, _DIM), lambda i: (0, 0)),
        pl.BlockSpec((1, _DIM), lambda i: (0, 0)),
        pl.BlockSpec((1, _DIM), lambda i: (0, 0)),
        pl.BlockSpec((_DIM, _DIM), lambda i: (0, 0)),
        pl.BlockSpec((1, _DIM), lambda i: (0, 0)),
        pl.BlockSpec((_DIM, _G), lambda i: (0, 0)),
        pl.BlockSpec((1, _G), lambda i: (0, 0)),
    ],
    out_specs=pl.BlockSpec((_G, _G), lambda i: (0, 0)),
    out_shape=jax.ShapeDtypeStruct((_G, _G), jnp.float32),
    scratch_shapes=[pltpu.VMEM((_G, _DIM), jnp.float32)],
)


def kernel(x, edge_index, batch, W1, b1, W2, b2, bn_g, bn_b,
           fc1_W, fc1_b, fc2_W, fc2_b):
    src = jnp.pad(edge_index[0].reshape(_NW, _EW),
                  ((0, 0), (0, _PAD))).reshape(_NW * _EWP)
    dst = jnp.pad(edge_index[1].reshape(_NW, _EW), ((0, 0), (0, _PAD)),
                  constant_values=_N).reshape(_NW, _NCHUNK, _C)
    agg2 = _edge_agg_fn()(src, dst, x)
    scale = (bn_g / jnp.sqrt(1.0 + 1e-5)).reshape(1, _DIM)
    shift = bn_b.reshape(1, _DIM)
    batch3 = batch.reshape(_NSTEP, 1, _BLK)
    f2wp = jnp.pad(fc2_W, ((0, 0), (0, _G - 1)))
    f2bp = jnp.pad(fc2_b, (0, _G - 1)).reshape(1, _G)
    outp = _tc_call(x, agg2, agg2, batch3, W1, b1.reshape(1, _DIM), W2,
                    b2.reshape(1, _DIM), scale, shift, fc1_W,
                    fc1_b.reshape(1, _DIM), f2wp, f2bp)
    return outp[:, :1]


# speedup vs baseline: 1.4797x; 1.4797x over previous
import functools

import jax
import jax.numpy as jnp
from jax import lax
from jax.experimental import pallas as pl
from jax.experimental.pallas import tpu as pltpu
from jax.experimental.pallas import tpu_sc as plsc

_N = 10000
_E = 320000
_DIN = 128
_DIM = 256
_G = 128

_NC = 2
_NS = 16
_NW = _NC * _NS
_EW = _E // _NW
_C = 80
_NCHUNK = _EW // _C
_RPT = 624
_TAIL = _N - _NS * _RPT

@functools.cache
def _edge_agg_fn():
    mesh = plsc.VectorSubcoreMesh(core_axis_name="c", subcore_axis_name="s")

    @functools.partial(
        pl.kernel,
        mesh=mesh,
        out_type=jax.ShapeDtypeStruct((_NC * _N, _DIN), jnp.float32),
        scratch_types=[
            pltpu.VMEM((_EW,), jnp.int32),
            pltpu.VMEM((_NCHUNK, _C), jnp.int32),
            pltpu.VMEM((_C, _DIN), jnp.float32),
            pltpu.VMEM((_C, _DIN), jnp.float32),
            pltpu.VMEM_SHARED((_N, _DIN), jnp.float32),
            pltpu.SemaphoreType.DMA,
            pltpu.SemaphoreType.DMA,
        ],
    )
    def _edge_agg(src_hbm, dst_hbm, x_hbm, out_hbm,
                  src_v, dst_v, rows0_v, rows1_v, agg_sh,
                  sem0, sem1):
        cid = lax.axis_index("c")
        sid = lax.axis_index("s")
        wid = cid * _NS + sid

        zvec = jnp.zeros((16,), jnp.float32)

        def _zfill(k, carry):
            r = k // (_DIN // 16)
            col = (k % (_DIN // 16)) * 16
            rows0_v[r, pl.ds(col, 16)] = zvec
            return carry

        lax.fori_loop(0, _C * (_DIN // 16), _zfill, 0)

        def _zcopy(k, carry):
            pltpu.sync_copy(rows0_v,
                            agg_sh.at[pl.ds(sid * _RPT + k * _C, _C)])
            return carry

        lax.fori_loop(0, _RPT // _C, _zcopy, 0)
        pltpu.sync_copy(rows0_v.at[pl.ds(0, _RPT - (_RPT // _C) * _C)],
                        agg_sh.at[pl.ds(sid * _RPT + (_RPT // _C) * _C,
                                        _RPT - (_RPT // _C) * _C)])

        @pl.when(sid == _NS - 1)
        def _ztail():
            pltpu.sync_copy(rows0_v.at[pl.ds(0, _TAIL)],
                            agg_sh.at[pl.ds(_NS * _RPT, _TAIL)])

        plsc.subcore_barrier()

        pltpu.sync_copy(src_hbm.at[pl.ds(wid * _EW, _EW)], src_v)
        pltpu.sync_copy(dst_hbm.at[wid], dst_v)

        pltpu.async_copy(x_hbm.at[src_v.at[pl.ds(0, _C)]], rows0_v, sem0)

        def _body(j, carry):
            @pl.when(j % 2 == 0)
            def _even():
                pltpu.make_async_copy(x_hbm.at[src_v.at[pl.ds(j * _C, _C)]],
                                      rows0_v, sem0).wait()

                @pl.when(j + 1 < _NCHUNK)
                def _pf():
                    pltpu.async_copy(
                        x_hbm.at[src_v.at[pl.ds((j + 1) * _C, _C)]],
                        rows1_v, sem1)

                pltpu.sync_copy(rows0_v, agg_sh.at[dst_v.at[j]], add=True)

            @pl.when(j % 2 == 1)
            def _odd():
                pltpu.make_async_copy(x_hbm.at[src_v.at[pl.ds(j * _C, _C)]],
                                      rows1_v, sem1).wait()

                @pl.when(j + 1 < _NCHUNK)
                def _pf():
                    pltpu.async_copy(
                        x_hbm.at[src_v.at[pl.ds((j + 1) * _C, _C)]],
                        rows0_v, sem0)

                pltpu.sync_copy(rows1_v, agg_sh.at[dst_v.at[j]], add=True)

            return carry

        lax.fori_loop(0, _NCHUNK, _body, 0)
        plsc.subcore_barrier()

        pltpu.sync_copy(agg_sh.at[pl.ds(sid * _RPT, _RPT)],
                        out_hbm.at[pl.ds(cid * _N + sid * _RPT, _RPT)])

        @pl.when(sid == _NS - 1)
        def _ptail():
            pltpu.sync_copy(agg_sh.at[pl.ds(_NS * _RPT, _TAIL)],
                            out_hbm.at[pl.ds(cid * _N + _NS * _RPT, _TAIL)])

    return _edge_agg


_BLK = 400
_NSTEP = _N // _BLK


def _tc_body(x_ref, a0_ref, a1_ref, batch_ref, w1_ref, b1_ref, w2_ref, b2_ref,
             sc_ref, sh_ref, f1w_ref, f1b_ref, f2w_ref, f2b_ref,
             out_ref, acc_ref):
    i = pl.program_id(0)
    h = x_ref[...] + a0_ref[...] + a1_ref[...]
    h1 = jnp.maximum(
        jnp.dot(h, w1_ref[...], preferred_element_type=jnp.float32)
        + b1_ref[...], 0.0)
    h2 = (jnp.dot(h1, w2_ref[...], preferred_element_type=jnp.float32)
          + b2_ref[...])
    h2 = jnp.maximum(h2, 0.0) * sc_ref[...] + sh_ref[...]

    b_row = batch_ref[0, :, :]
    ohT = (jnp.broadcast_to(b_row, (_G, _BLK))
           == lax.broadcasted_iota(jnp.int32, (_G, _BLK), 0)
           ).astype(jnp.float32)
    pooled = jnp.dot(ohT, h2, preferred_element_type=jnp.float32)

    @pl.when(i == 0)
    def _init():
        acc_ref[...] = jnp.zeros_like(acc_ref)

    acc_ref[...] += pooled

    @pl.when(i == _NSTEP - 1)
    def _head():
        g = acc_ref[...]
        gg = jnp.maximum(
            jnp.dot(g, f1w_ref[...], preferred_element_type=jnp.float32)
            + f1b_ref[...], 0.0)
        out_ref[...] = (jnp.dot(gg, f2w_ref[...],
                                preferred_element_type=jnp.float32)
                        + f2b_ref[...])


_tc_call = pl.pallas_call(
    _tc_body,
    grid=(_NSTEP,),
    in_specs=[
        pl.BlockSpec((_BLK, _DIN), lambda i: (i, 0)),
        pl.BlockSpec((_BLK, _DIN), lambda i: (i, 0)),
        pl.BlockSpec((_BLK, _DIN),
                     lambda i: (i + _N // _BLK, 0)),
        pl.BlockSpec((1, 1, _BLK), lambda i: (i, 0, 0)),
        pl.BlockSpec((_DIN, _DIM), lambda i: (0, 0)),
        pl.BlockSpec((1, _DIM), lambda i: (0, 0)),
        pl.BlockSpec((_DIM, _DIM), lambda i: (0, 0)),
        pl.BlockSpec((1, _DIM), lambda i: (0, 0)),
        pl.BlockSpec((1, _DIM), lambda i: (0, 0)),
        pl.BlockSpec((1, _DIM), lambda i: (0, 0)),
        pl.BlockSpec((_DIM, _DIM), lambda i: (0, 0)),
        pl.BlockSpec((1, _DIM), lambda i: (0, 0)),
        pl.BlockSpec((_DIM, _G), lambda i: (0, 0)),
        pl.BlockSpec((1, _G), lambda i: (0, 0)),
    ],
    out_specs=pl.BlockSpec((_G, _G), lambda i: (0, 0)),
    out_shape=jax.ShapeDtypeStruct((_G, _G), jnp.float32),
    scratch_shapes=[pltpu.VMEM((_G, _DIM), jnp.float32)],
)


def kernel(x, edge_index, batch, W1, b1, W2, b2, bn_g, bn_b,
           fc1_W, fc1_b, fc2_W, fc2_b):
    src = edge_index[0]
    dst = edge_index[1].reshape(_NW, _NCHUNK, _C)
    agg2 = _edge_agg_fn()(src, dst, x)
    scale = (bn_g / jnp.sqrt(1.0 + 1e-5)).reshape(1, _DIM)
    shift = bn_b.reshape(1, _DIM)
    batch3 = batch.reshape(_NSTEP, 1, _BLK)
    f2wp = jnp.pad(fc2_W, ((0, 0), (0, _G - 1)))
    f2bp = jnp.pad(fc2_b, (0, _G - 1)).reshape(1, _G)
    outp = _tc_call(x, agg2, agg2, batch3, W1, b1.reshape(1, _DIM), W2,
                    b2.reshape(1, _DIM), scale, shift, fc1_W,
                    fc1_b.reshape(1, _DIM), f2wp, f2bp)
    return outp[:, :1]


# speedup vs baseline: 1.5015x; 1.0148x over previous
import functools

import jax
import jax.numpy as jnp
from jax import lax
from jax.experimental import pallas as pl
from jax.experimental.pallas import tpu as pltpu
from jax.experimental.pallas import tpu_sc as plsc

_N = 10000
_E = 320000
_DIN = 128
_DIM = 256
_G = 128

_NC = 2
_NS = 16
_NW = _NC * _NS
_EW = _E // _NW
_C = 80
_NCHUNK = _EW // _C
_RPT = 624
_TAIL = _N - _NS * _RPT

@functools.cache
def _edge_agg_fn():
    mesh = plsc.VectorSubcoreMesh(core_axis_name="c", subcore_axis_name="s")

    @functools.partial(
        pl.kernel,
        mesh=mesh,
        out_type=jax.ShapeDtypeStruct((_NC * _N, _DIN), jnp.float32),
        scratch_types=[
            pltpu.VMEM((_EW,), jnp.int32),
            pltpu.VMEM((_NCHUNK, _C), jnp.int32),
            pltpu.VMEM((_C, _DIN), jnp.float32),
            pltpu.VMEM((_C, _DIN), jnp.float32),
            pltpu.VMEM_SHARED((_N, _DIN), jnp.float32),
            pltpu.SemaphoreType.DMA,
            pltpu.SemaphoreType.DMA,
            pltpu.SemaphoreType.DMA,
        ],
    )
    def _edge_agg(src_hbm, dst_hbm, x_hbm, out_hbm,
                  src_v, dst_v, rows0_v, rows1_v, agg_sh,
                  sem0, sem1, semz):
        cid = lax.axis_index("c")
        sid = lax.axis_index("s")
        wid = cid * _NS + sid
        _ZREM = _RPT - (_RPT // _C) * _C

        pltpu.async_copy(src_hbm.at[pl.ds(wid * _EW, _EW)], src_v, sem0)
        pltpu.async_copy(dst_hbm.at[wid], dst_v, sem1)

        zvec = jnp.zeros((16,), jnp.float32)

        def _zfill(k, carry):
            r = k // (_DIN // 16)
            col = (k % (_DIN // 16)) * 16
            rows0_v[r, pl.ds(col, 16)] = zvec
            return carry

        lax.fori_loop(0, _C * (_DIN // 16), _zfill, 0)

        def _zcopy(k, carry):
            pltpu.async_copy(rows0_v,
                             agg_sh.at[pl.ds(sid * _RPT + k * _C, _C)],
                             semz)
            return carry

        lax.fori_loop(0, _RPT // _C, _zcopy, 0)
        pltpu.async_copy(rows0_v.at[pl.ds(0, _ZREM)],
                         agg_sh.at[pl.ds(sid * _RPT + (_RPT // _C) * _C,
                                         _ZREM)], semz)

        @pl.when(sid == _NS - 1)
        def _ztail():
            pltpu.async_copy(rows0_v.at[pl.ds(0, _TAIL)],
                             agg_sh.at[pl.ds(_NS * _RPT, _TAIL)], semz)

        def _zdrain(k, carry):
            pltpu.make_async_copy(
                rows0_v, agg_sh.at[pl.ds(sid * _RPT + k * _C, _C)],
                semz).wait()
            return carry

        lax.fori_loop(0, _RPT // _C, _zdrain, 0)
        pltpu.make_async_copy(
            rows0_v.at[pl.ds(0, _ZREM)],
            agg_sh.at[pl.ds(sid * _RPT + (_RPT // _C) * _C, _ZREM)],
            semz).wait()

        @pl.when(sid == _NS - 1)
        def _ztaildrain():
            pltpu.make_async_copy(rows0_v.at[pl.ds(0, _TAIL)],
                                  agg_sh.at[pl.ds(_NS * _RPT, _TAIL)],
                                  semz).wait()

        plsc.subcore_barrier()

        pltpu.make_async_copy(src_hbm.at[pl.ds(wid * _EW, _EW)],
                              src_v, sem0).wait()
        pltpu.make_async_copy(dst_hbm.at[wid], dst_v, sem1).wait()

        pltpu.async_copy(x_hbm.at[src_v.at[pl.ds(0, _C)]], rows0_v, sem0)

        def _body(j, carry):
            @pl.when(j % 2 == 0)
            def _even():
                pltpu.make_async_copy(x_hbm.at[src_v.at[pl.ds(j * _C, _C)]],
                                      rows0_v, sem0).wait()

                @pl.when(j + 1 < _NCHUNK)
                def _pf():
                    pltpu.async_copy(
                        x_hbm.at[src_v.at[pl.ds((j + 1) * _C, _C)]],
                        rows1_v, sem1)

                pltpu.sync_copy(rows0_v, agg_sh.at[dst_v.at[j]], add=True)

            @pl.when(j % 2 == 1)
            def _odd():
                pltpu.make_async_copy(x_hbm.at[src_v.at[pl.ds(j * _C, _C)]],
                                      rows1_v, sem1).wait()

                @pl.when(j + 1 < _NCHUNK)
                def _pf():
                    pltpu.async_copy(
                        x_hbm.at[src_v.at[pl.ds((j + 1) * _C, _C)]],
                        rows0_v, sem0)

                pltpu.sync_copy(rows1_v, agg_sh.at[dst_v.at[j]], add=True)

            return carry

        lax.fori_loop(0, _NCHUNK, _body, 0)
        plsc.subcore_barrier()

        pltpu.sync_copy(agg_sh.at[pl.ds(sid * _RPT, _RPT)],
                        out_hbm.at[pl.ds(cid * _N + sid * _RPT, _RPT)])

        @pl.when(sid == _NS - 1)
        def _ptail():
            pltpu.sync_copy(agg_sh.at[pl.ds(_NS * _RPT, _TAIL)],
                            out_hbm.at[pl.ds(cid * _N + _NS * _RPT, _TAIL)])

    return _edge_agg


_BLK = 400
_NSTEP = _N // _BLK


def _tc_body(x_ref, a0_ref, a1_ref, batch_ref, w1_ref, b1_ref, w2_ref, b2_ref,
             sc_ref, sh_ref, f1w_ref, f1b_ref, f2w_ref, f2b_ref,
             out_ref, acc_ref):
    i = pl.program_id(0)
    h = x_ref[...] + a0_ref[...] + a1_ref[...]
    h1 = jnp.maximum(
        jnp.dot(h, w1_ref[...], preferred_element_type=jnp.float32)
        + b1_ref[...], 0.0)
    h2 = (jnp.dot(h1, w2_ref[...], preferred_element_type=jnp.float32)
          + b2_ref[...])
    h2 = jnp.maximum(h2, 0.0) * sc_ref[...] + sh_ref[...]

    b_row = batch_ref[0, :, :]
    ohT = (jnp.broadcast_to(b_row, (_G, _BLK))
           == lax.broadcasted_iota(jnp.int32, (_G, _BLK), 0)
           ).astype(jnp.float32)
    pooled = jnp.dot(ohT, h2, preferred_element_type=jnp.float32)

    @pl.when(i == 0)
    def _init():
        acc_ref[...] = jnp.zeros_like(acc_ref)

    acc_ref[...] += pooled

    @pl.when(i == _NSTEP - 1)
    def _head():
        g = acc_ref[...]
        gg = jnp.maximum(
            jnp.dot(g, f1w_ref[...], preferred_element_type=jnp.float32)
            + f1b_ref[...], 0.0)
        out_ref[...] = (jnp.dot(gg, f2w_ref[...],
                                preferred_element_type=jnp.float32)
                        + f2b_ref[...])


_tc_call = pl.pallas_call(
    _tc_body,
    grid=(_NSTEP,),
    in_specs=[
        pl.BlockSpec((_BLK, _DIN), lambda i: (i, 0)),
        pl.BlockSpec((_BLK, _DIN), lambda i: (i, 0)),
        pl.BlockSpec((_BLK, _DIN),
                     lambda i: (i + _N // _BLK, 0)),
        pl.BlockSpec((1, 1, _BLK), lambda i: (i, 0, 0)),
        pl.BlockSpec((_DIN, _DIM), lambda i: (0, 0)),
        pl.BlockSpec((1, _DIM), lambda i: (0, 0)),
        pl.BlockSpec((_DIM, _DIM), lambda i: (0, 0)),
        pl.BlockSpec((1, _DIM), lambda i: (0, 0)),
        pl.BlockSpec((1, _DIM), lambda i: (0, 0)),
        pl.BlockSpec((1, _DIM), lambda i: (0, 0)),
        pl.BlockSpec((_DIM, _DIM), lambda i: (0, 0)),
        pl.BlockSpec((1, _DIM), lambda i: (0, 0)),
        pl.BlockSpec((_DIM, _G), lambda i: (0, 0)),
        pl.BlockSpec((1, _G), lambda i: (0, 0)),
    ],
    out_specs=pl.BlockSpec((_G, _G), lambda i: (0, 0)),
    out_shape=jax.ShapeDtypeStruct((_G, _G), jnp.float32),
    scratch_shapes=[pltpu.VMEM((_G, _DIM), jnp.float32)],
)


def kernel(x, edge_index, batch, W1, b1, W2, b2, bn_g, bn_b,
           fc1_W, fc1_b, fc2_W, fc2_b):
    src = edge_index[0]
    dst = edge_index[1].reshape(_NW, _NCHUNK, _C)
    agg2 = _edge_agg_fn()(src, dst, x)
    scale = (bn_g / jnp.sqrt(1.0 + 1e-5)).reshape(1, _DIM)
    shift = bn_b.reshape(1, _DIM)
    batch3 = batch.reshape(_NSTEP, 1, _BLK)
    f2wp = jnp.pad(fc2_W, ((0, 0), (0, _G - 1)))
    f2bp = jnp.pad(fc2_b, (0, _G - 1)).reshape(1, _G)
    outp = _tc_call(x, agg2, agg2, batch3, W1, b1.reshape(1, _DIM), W2,
                    b2.reshape(1, _DIM), scale, shift, fc1_W,
                    fc1_b.reshape(1, _DIM), f2wp, f2bp)
    return outp[:, :1]
